# baseline (speedup 1.0000x reference)
import functools

import jax
import jax.numpy as jnp
from jax.experimental import pallas as pl
from jax.experimental.pallas import tpu as pltpu


def _gcn3_kernel(a_ref, x_ref, w1_ref, w2_ref, w3_ref, b1_ref, b2_ref, b3_ref,
                 o_ref, a_bf_ref, z_ref, h1_ref, h2_ref, *, tm, out_dim):
    l = pl.program_id(0)
    i = pl.program_id(1)
    f32 = jnp.float32
    bf = jnp.bfloat16
    hm = tm // 2
    n = a_bf_ref.shape[0]
    nh = n // 2

    @pl.when((l == 0) & (i == 0))
    def _():
        w1b = w1_ref[...]
        z_ref[:nh, :] = jnp.dot(x_ref[:nh, :].astype(bf), w1b,
                                preferred_element_type=f32).astype(bf)
        z_ref[nh:, :] = jnp.dot(x_ref[nh:, :].astype(bf), w1b,
                                preferred_element_type=f32).astype(bf)

    @pl.when((l == 1) & (i == 0))
    def _():
        w2b = w2_ref[...]
        z_ref[:nh, :] = jnp.dot(h1_ref[:nh, :], w2b,
                                preferred_element_type=f32).astype(bf)
        z_ref[nh:, :] = jnp.dot(h1_ref[nh:, :], w2b,
                                preferred_element_type=f32).astype(bf)

    @pl.when((l == 2) & (i == 0))
    def _():
        w3b = w3_ref[...]
        z_ref[:nh, :] = jnp.dot(h2_ref[:nh, :], w3b,
                                preferred_element_type=f32).astype(bf)
        z_ref[nh:, :] = jnp.dot(h2_ref[nh:, :], w3b,
                                preferred_element_type=f32).astype(bf)

    rows = pl.ds(i * tm, tm)
    nb = pl.num_programs(1) - 1

    @pl.when((l == 0) & (i < nb))
    def _():
        a_bf_ref[rows, :] = a_ref[...].astype(bf)

    def _agg(base, out_ref, bias, relu, width):
        z = z_ref[...]
        for h in range(2):
            r = pl.ds(base + h * hm, hm)
            acc = jnp.dot(a_bf_ref[r, :], z, preferred_element_type=f32)
            v = acc[:, :width] + bias
            if relu:
                v = jnp.maximum(v, 0.0).astype(bf)
            out_ref[r, :] = v

    @pl.when((l == 0) & (i > 0))
    def _():
        _agg((i - 1) * tm, h1_ref, b1_ref[...], True, z_ref.shape[1])

    @pl.when((l == 1) & (i < nb))
    def _():
        _agg(i * tm, h2_ref, b2_ref[...], True, z_ref.shape[1])

    @pl.when((l == 2) & (i < nb))
    def _():
        _agg(i * tm, o_ref, b3_ref[...], False, out_dim)


def kernel(a_hat, x, w1, b1, w2, b2, w3, b3):
    n = a_hat.shape[0]
    in_dim = x.shape[1]
    hid1 = w1.shape[1]
    hid2 = w2.shape[1]
    out_dim = w3.shape[1]

    tm = min(256, n)
    n_blocks = n // tm
    zw = max(hid1, hid2, out_dim)
    bf = jnp.bfloat16

    def padw(w):
        return jnp.pad(w.astype(bf), ((0, zw - w.shape[0]),
                                      (0, zw - w.shape[1])))

    w1p = padw(w1) if (w1.shape[0] < zw or hid1 < zw) else w1.astype(bf)
    w2p = padw(w2)
    w3p = padw(w3)
    b1p = jnp.pad(b1.reshape(1, -1), ((0, 0), (0, zw - hid1)))
    b2p = jnp.pad(b2.reshape(1, -1), ((0, 0), (0, zw - hid2)))

    body = functools.partial(_gcn3_kernel, tm=tm, out_dim=out_dim)

    return pl.pallas_call(
        body,
        out_shape=jax.ShapeDtypeStruct((n, out_dim), jnp.float32),
        grid=(3, n_blocks + 1),
        in_specs=[
            pl.BlockSpec((tm, n),
                         lambda l, i: (jnp.where(l == 0,
                                                 jnp.minimum(i, n_blocks - 1),
                                                 n_blocks - 1), 0)),
            pl.BlockSpec((n, in_dim), lambda l, i: (0, 0)),
            pl.BlockSpec((in_dim, zw), lambda l, i: (0, 0)),
            pl.BlockSpec((zw, zw), lambda l, i: (0, 0)),
            pl.BlockSpec((zw, zw), lambda l, i: (0, 0)),
            pl.BlockSpec((1, zw), lambda l, i: (0, 0)),
            pl.BlockSpec((1, zw), lambda l, i: (0, 0)),
            pl.BlockSpec((1, out_dim), lambda l, i: (0, 0)),
        ],
        out_specs=pl.BlockSpec((n, out_dim), lambda l, i: (0, 0)),
        scratch_shapes=[
            pltpu.VMEM((n, n), bf),
            pltpu.VMEM((n, zw), bf),
            pltpu.VMEM((n, zw), bf),
            pltpu.VMEM((n, zw), bf),
        ],
        compiler_params=pltpu.CompilerParams(
            dimension_semantics=("arbitrary", "arbitrary"),
            vmem_limit_bytes=60 << 20,
        ),
    )(a_hat, x, w1p, w2p, w3p, b1p, b2p, b3.reshape(1, -1))
